# Initial kernel scaffold; baseline (speedup 1.0000x reference)
import jax
import jax.numpy as jnp
from jax.experimental import pallas as pl


def kernel(x, y, prototypes, omega):
    raise NotImplementedError("write your pallas kernel here")


# speedup vs baseline: 8.6141x; 8.6141x over previous
import jax
import jax.numpy as jnp
from jax.experimental import pallas as pl
from jax.experimental.pallas import tpu as pltpu

_B, _D, _C, _P = 1024, 256, 8, 512
_PC = _P // _C
_LAM = 1.0


def _glmvq_body(y_ref, x_ref, pg_ref, om_ref, out_ref):
    x = x_ref[...]
    yrow = y_ref[...]
    pos = jnp.zeros((1, _B), jnp.float32)
    neg = jnp.full((1, _B), jnp.inf, jnp.float32)
    for c in range(_C):
        om_c = om_ref[c]
        txT = jax.lax.dot_general(om_c, x, (((1,), (1,)), ((), ())),
                                  preferred_element_type=jnp.float32)
        ntx = jnp.sum(txT * txT, axis=0, keepdims=True)
        pc = pg_ref[c]
        tp = jax.lax.dot_general(pc, om_c, (((1,), (1,)), ((), ())),
                                 preferred_element_type=jnp.float32)
        ntp = jnp.sum(tp * tp, axis=1, keepdims=True)
        crossT = jax.lax.dot_general(tp, txT, (((1,), (0,)), ((), ())),
                                     preferred_element_type=jnp.float32)
        dmin = jnp.min(ntp - 2.0 * crossT, axis=0, keepdims=True) + ntx
        is_c = yrow == c
        pos = pos + jnp.where(is_c, dmin, 0.0)
        neg = jnp.minimum(neg, jnp.where(is_c, jnp.inf, dmin))
    mu = (pos - neg) / (pos + neg)
    sig = 1.0 / (1.0 + jnp.exp(-_LAM * mu))
    om = om_ref[...]
    reg = jnp.sqrt(jnp.sum(om * om))
    out_ref[0, 0] = jnp.sum(sig) / _B + 0.01 * reg


def kernel(x, y, prototypes, omega):
    pg = prototypes.reshape(_PC, _C, _D).transpose(1, 0, 2)
    y_row = y.reshape(1, _B)
    out = pl.pallas_call(
        _glmvq_body,
        out_shape=jax.ShapeDtypeStruct((1, 1), jnp.float32),
        out_specs=pl.BlockSpec(memory_space=pltpu.SMEM),
    )(y_row, x, pg, omega)
    return out[0, 0]
